# baseline (speedup 1.0000x reference)
import functools

import jax
import jax.numpy as jnp
from jax import lax
from jax.experimental import pallas as pl
from jax.experimental.pallas import tpu as pltpu
from jax.experimental.pallas import tpu_sc as plsc

_NC = 2
_NS = 16
_NW = _NC * _NS
_C = 32
_K = 16


@functools.lru_cache(maxsize=None)
def _make_gather(B, V, D):
    assert B % (_NW * _C * _K) == 0
    rpw = B // (_NW * _C)
    nblk = rpw // _K
    cb = _C * _K

    mesh = plsc.VectorSubcoreMesh(core_axis_name="c", subcore_axis_name="s")

    @functools.partial(
        pl.kernel,
        out_type=jax.ShapeDtypeStruct((B, D), jnp.float32),
        mesh=mesh,
        scratch_types=[
            pltpu.VMEM((rpw, _C), jnp.int32),
            pltpu.VMEM((2, cb, D), jnp.float32),
            pltpu.SemaphoreType.DMA,
        ],
        compiler_params=pltpu.CompilerParams(use_tc_tiling_on_sc=False),
    )
    def gather_kernel(x_hbm, table_hbm, out_hbm, idx_v, rows_v, sem_g):
        wid = lax.axis_index("s") * _NC + lax.axis_index("c")
        base = wid * rpw * _C
        pltpu.sync_copy(x_hbm.at[pl.ds(wid * rpw, rpw)], idx_v)

        def fire(i, buf):
            for j in range(_K):
                pltpu.make_async_copy(
                    table_hbm.at[idx_v.at[i * _K + j]],
                    rows_v.at[buf].at[pl.ds(j * _C, _C)],
                    sem_g,
                ).start()

        def drain(i, buf):
            for j in range(_K):
                pltpu.make_async_copy(
                    table_hbm.at[idx_v.at[i * _K + j]],
                    rows_v.at[buf].at[pl.ds(j * _C, _C)],
                    sem_g,
                ).wait()

        fire(0, 0)

        def blk(i, carry):
            cur = lax.rem(i, 2)

            @pl.when(i + 1 < nblk)
            def _():
                fire(i + 1, 1 - cur)

            drain(i, cur)
            pltpu.sync_copy(
                rows_v.at[cur], out_hbm.at[pl.ds(base + i * cb, cb)]
            )
            return carry

        lax.fori_loop(0, nblk, blk, 0)

    return gather_kernel


def kernel(x, table):
    b, h = x.shape
    v, d = table.shape
    flat = b * h
    xc = jnp.maximum(x.reshape(flat // _C, _C), 0)
    out = _make_gather(flat, v, d)(xc, table)
    return out.reshape(b, h, d) + jnp.float32(1e-45)


# speedup vs baseline: 1.8824x; 1.8824x over previous
import functools

import jax
import jax.numpy as jnp
from jax import lax
from jax.experimental import pallas as pl
from jax.experimental.pallas import tpu as pltpu
from jax.experimental.pallas import tpu_sc as plsc

_NC = 2
_NS = 16
_NW = _NC * _NS
_RB = 16


@functools.lru_cache(maxsize=None)
def _make_gather(B, H, V, D):
    assert B % _NW == 0
    bpw = B // _NW
    assert bpw % _RB == 0
    nblk = bpw // _RB

    mesh = plsc.VectorSubcoreMesh(core_axis_name="c", subcore_axis_name="s")

    @functools.partial(
        pl.kernel,
        out_type=jax.ShapeDtypeStruct((B, H, D), jnp.float32),
        mesh=mesh,
        scratch_types=[
            pltpu.VMEM((bpw, H), jnp.int32),
            pltpu.VMEM((2, _RB, H, D), jnp.float32),
            pltpu.SemaphoreType.DMA,
        ],
        compiler_params=pltpu.CompilerParams(use_tc_tiling_on_sc=False),
    )
    def gather_kernel(x_hbm, table_hbm, out_hbm, idx_v, rows_v, sem_g):
        wid = lax.axis_index("s") * _NC + lax.axis_index("c")
        xbase = wid * bpw
        pltpu.sync_copy(x_hbm.at[pl.ds(xbase, bpw)], idx_v)

        def fire(i, buf):
            for j in range(_RB):
                pltpu.make_async_copy(
                    table_hbm.at[idx_v.at[i * _RB + j]],
                    rows_v.at[buf].at[j],
                    sem_g,
                ).start()

        def drain(i, buf):
            for j in range(_RB):
                pltpu.make_async_copy(
                    table_hbm.at[idx_v.at[i * _RB + j]],
                    rows_v.at[buf].at[j],
                    sem_g,
                ).wait()

        fire(0, 0)

        def blk(i, carry):
            cur = lax.rem(i, 2)

            @pl.when(i + 1 < nblk)
            def _():
                fire(i + 1, 1 - cur)

            drain(i, cur)
            pltpu.sync_copy(
                rows_v.at[cur], out_hbm.at[pl.ds(xbase + i * _RB, _RB)]
            )
            return carry

        lax.fori_loop(0, nblk, blk, 0)

    return gather_kernel


def kernel(x, table):
    b, h = x.shape
    v, d = table.shape
    return _make_gather(b, h, v, d)(x, table)
